# Initial kernel scaffold; baseline (speedup 1.0000x reference)
import jax
import jax.numpy as jnp
from jax.experimental import pallas as pl


def kernel(buckets, node_ids, token_ids):
    raise NotImplementedError("write your pallas kernel here")


# speedup vs baseline: 6.3218x; 6.3218x over previous
import functools

import jax
import jax.numpy as jnp
from jax import lax
from jax.experimental import pallas as pl
from jax.experimental.pallas import tpu as pltpu
from jax.experimental.pallas import tpu_sc as plsc

EMB = 128
TOKENS = 20
LANES = 16
CHUNKS = EMB // LANES

NODES_PER_GATHER = 4
IDX_PER_GATHER = NODES_PER_GATHER * TOKENS
GATHERS_PER_STEP = 4
NODES_PER_STEP = NODES_PER_GATHER * GATHERS_PER_STEP
ROWS_PER_STEP = NODES_PER_STEP * TOKENS


def _build_sc_kernel(batch, n_workers):
    nodes_per_tile = batch // n_workers
    steps = nodes_per_tile // NODES_PER_STEP
    idx_rows_per_tile = nodes_per_tile // NODES_PER_GATHER

    mesh = plsc.VectorSubcoreMesh(core_axis_name="c", subcore_axis_name="s")
    nc = 2

    @functools.partial(
        pl.kernel,
        mesh=mesh,
        out_type=jax.ShapeDtypeStruct((batch, EMB), jnp.float32),
        scratch_types=[
            pltpu.VMEM((idx_rows_per_tile, IDX_PER_GATHER), jnp.int32),
            pltpu.VMEM((ROWS_PER_STEP, EMB), jnp.float32),
            pltpu.VMEM((NODES_PER_STEP, EMB), jnp.float32),
            pltpu.SemaphoreType.DMA,
        ],
    )
    def emb_kernel(tok_hbm, table_hbm, out_hbm, idx_v, rows_v, out_v, sem):
        i32 = lambda v: jnp.int32(v)
        wid = lax.axis_index("s") * i32(nc) + lax.axis_index("c")
        idx_row0 = wid * i32(idx_rows_per_tile)
        node0 = wid * i32(nodes_per_tile)

        pltpu.sync_copy(tok_hbm.at[pl.ds(idx_row0, idx_rows_per_tile)], idx_v)

        def step(s, carry):
            for j in range(GATHERS_PER_STEP):
                pltpu.async_copy(
                    table_hbm.at[idx_v.at[s * i32(GATHERS_PER_STEP) + i32(j)]],
                    rows_v.at[pl.ds(j * IDX_PER_GATHER, IDX_PER_GATHER)],
                    sem,
                )
            for j in range(GATHERS_PER_STEP):
                pltpu.make_async_copy(
                    table_hbm.at[idx_v.at[s * i32(GATHERS_PER_STEP) + i32(j)]],
                    rows_v.at[pl.ds(j * IDX_PER_GATHER, IDX_PER_GATHER)],
                    sem,
                ).wait()

            def node(g, carry2):
                base = g * i32(TOKENS)
                for c in range(CHUNKS):
                    sl = pl.ds(c * LANES, LANES)
                    acc = rows_v[base, sl]
                    for t in range(1, TOKENS):
                        acc = acc + rows_v[base + i32(t), sl]
                    out_v[g, sl] = acc
                return carry2

            lax.fori_loop(0, jnp.int32(NODES_PER_STEP), node, 0, unroll=False)

            pltpu.sync_copy(
                out_v,
                out_hbm.at[pl.ds(node0 + s * i32(NODES_PER_STEP), NODES_PER_STEP)],
            )
            return carry

        lax.fori_loop(0, jnp.int32(steps), step, 0, unroll=False)

    return emb_kernel


def kernel(buckets, node_ids, token_ids):
    del node_ids
    batch = token_ids.shape[0]
    n_workers = 32
    tok = token_ids.astype(jnp.int32).reshape(
        batch * TOKENS // IDX_PER_GATHER, IDX_PER_GATHER
    )
    emb_kernel = _build_sc_kernel(batch, n_workers)
    return emb_kernel(tok, buckets)


# speedup vs baseline: 9.3690x; 1.4820x over previous
import functools

import jax
import jax.numpy as jnp
from jax import lax
from jax.experimental import pallas as pl
from jax.experimental.pallas import tpu as pltpu
from jax.experimental.pallas import tpu_sc as plsc

EMB = 128
TOKENS = 20
LANES = 16
CHUNKS = EMB // LANES

NODES_PER_GATHER = 4
IDX_PER_GATHER = NODES_PER_GATHER * TOKENS
GATHERS_PER_STEP = 4
NODES_PER_STEP = NODES_PER_GATHER * GATHERS_PER_STEP
ROWS_PER_STEP = NODES_PER_STEP * TOKENS


def _build_sc_kernel(batch, n_workers):
    nodes_per_tile = batch // n_workers
    steps = nodes_per_tile // NODES_PER_STEP
    idx_rows_per_tile = nodes_per_tile // NODES_PER_GATHER

    mesh = plsc.VectorSubcoreMesh(core_axis_name="c", subcore_axis_name="s")
    nc = 2

    @functools.partial(
        pl.kernel,
        mesh=mesh,
        out_type=jax.ShapeDtypeStruct((batch, EMB), jnp.float32),
        scratch_types=[
            pltpu.VMEM((idx_rows_per_tile, IDX_PER_GATHER), jnp.int32),
            pltpu.VMEM((ROWS_PER_STEP, EMB), jnp.float32),
            pltpu.VMEM((ROWS_PER_STEP, EMB), jnp.float32),
            pltpu.VMEM((NODES_PER_STEP, EMB), jnp.float32),
            pltpu.SemaphoreType.DMA,
            pltpu.SemaphoreType.DMA,
        ],
    )
    def emb_kernel(tok_hbm, table_hbm, out_hbm, idx_v, rows_a, rows_b, out_v,
                   sem_a, sem_b):
        i32 = lambda v: jnp.int32(v)
        wid = lax.axis_index("s") * i32(nc) + lax.axis_index("c")
        idx_row0 = wid * i32(idx_rows_per_tile)
        node0 = wid * i32(nodes_per_tile)

        pltpu.sync_copy(tok_hbm.at[pl.ds(idx_row0, idx_rows_per_tile)], idx_v)

        def fire(s, rows_v, sem):
            for j in range(GATHERS_PER_STEP):
                pltpu.async_copy(
                    table_hbm.at[idx_v.at[s * i32(GATHERS_PER_STEP) + i32(j)]],
                    rows_v.at[pl.ds(j * IDX_PER_GATHER, IDX_PER_GATHER)],
                    sem,
                )

        def drain(s, rows_v, sem):
            for j in range(GATHERS_PER_STEP):
                pltpu.make_async_copy(
                    table_hbm.at[idx_v.at[s * i32(GATHERS_PER_STEP) + i32(j)]],
                    rows_v.at[pl.ds(j * IDX_PER_GATHER, IDX_PER_GATHER)],
                    sem,
                ).wait()

        def compute(s, rows_v):
            def node(g, carry2):
                base = g * i32(TOKENS)
                for c in range(CHUNKS):
                    sl = pl.ds(c * LANES, LANES)
                    acc = rows_v[base, sl]
                    for t in range(1, TOKENS):
                        acc = acc + rows_v[base + i32(t), sl]
                    out_v[g, sl] = acc
                return carry2

            lax.fori_loop(0, jnp.int32(NODES_PER_STEP), node, 0, unroll=False)
            pltpu.sync_copy(
                out_v,
                out_hbm.at[pl.ds(node0 + s * i32(NODES_PER_STEP), NODES_PER_STEP)],
            )

        fire(i32(0), rows_a, sem_a)

        def pair(p, carry):
            s0 = p * i32(2)
            s1 = s0 + i32(1)
            fire(s1, rows_b, sem_b)
            drain(s0, rows_a, sem_a)
            compute(s0, rows_a)

            @pl.when(p < i32(steps // 2 - 1))
            def _():
                fire(s1 + i32(1), rows_a, sem_a)

            drain(s1, rows_b, sem_b)
            compute(s1, rows_b)
            return carry

        lax.fori_loop(0, jnp.int32(steps // 2), pair, 0, unroll=False)

    return emb_kernel


def kernel(buckets, node_ids, token_ids):
    del node_ids
    batch = token_ids.shape[0]
    n_workers = 32
    tok = token_ids.astype(jnp.int32).reshape(
        batch * TOKENS // IDX_PER_GATHER, IDX_PER_GATHER
    )
    emb_kernel = _build_sc_kernel(batch, n_workers)
    return emb_kernel(tok, buckets)
